# baseline (speedup 1.0000x reference)
import functools

import jax
import jax.numpy as jnp
from jax import lax
from jax.experimental import pallas as pl
from jax.experimental.pallas import tpu as pltpu
from jax.experimental.pallas import tpu_sc as plsc

N = 10000
E = 320000
D = 128

NC = 2
NS = 16
NW = NC * NS
LANES = 16

CHUNK = 128
CPW = 80
PW = CPW * CHUNK
EP = PW * NW

RPS = 632
NP = RPS * NS

_mesh = plsc.VectorSubcoreMesh(core_axis_name="c", subcore_axis_name="s")
_sc_params = pltpu.CompilerParams(use_tc_tiling_on_sc=False)


def _sc_agg_body(with_deg, x_hbm, pidx_hbm, *refs):
    if with_deg:
        (agg_out, deg_out, pidx0, pidx1, sidx0, sidx1, didx0, didx1,
         rows0, rows1, ones_v, zdeg, agg_sh, deg_sh,
         sem0, sem1, psem0, psem1) = refs
    else:
        (agg_out, pidx0, pidx1, sidx0, sidx1, didx0, didx1,
         rows0, rows1, agg_sh, sem0, sem1, psem0, psem1) = refs

    cid = lax.axis_index("c")
    sid = lax.axis_index("s")
    wid = cid * NS + sid

    zero16 = jnp.zeros((LANES,), jnp.float32)

    @pl.loop(0, CHUNK)
    def _(r):
        @pl.loop(0, D // LANES)
        def _(g):
            rows0[r, pl.ds(g * LANES, LANES)] = zero16

    if with_deg:
        @pl.loop(0, zdeg.shape[0])
        def _(r):
            zdeg[r, pl.ds(0, LANES)] = zero16

        ones16 = jnp.ones((LANES,), jnp.float32)

        @pl.loop(0, CHUNK)
        def _(r):
            ones_v[r, pl.ds(0, LANES)] = ones16

    r0 = sid * RPS
    nfull, rem = RPS // CHUNK, RPS % CHUNK

    for j in range(nfull):
        pltpu.sync_copy(rows0, agg_sh.at[pl.ds(r0 + j * CHUNK, CHUNK)])
    if rem:
        pltpu.sync_copy(rows0.at[pl.ds(0, rem)],
                        agg_sh.at[pl.ds(r0 + nfull * CHUNK, rem)])
    if with_deg:
        zn, zrem = RPS // zdeg.shape[0], RPS % zdeg.shape[0]
        for j in range(zn):
            pltpu.sync_copy(
                zdeg, deg_sh.at[pl.ds(r0 + j * zdeg.shape[0],
                                      zdeg.shape[0])])
        if zrem:
            pltpu.sync_copy(zdeg.at[pl.ds(0, zrem)],
                            deg_sh.at[pl.ds(r0 + zn * zdeg.shape[0], zrem)])

    plsc.subcore_barrier()

    def _pload(i, pidx, psem):
        return pltpu.make_async_copy(pidx_hbm.at[wid, i], pidx, psem)

    def _gather(sidx, rows, sem):
        return pltpu.make_async_copy(x_hbm.at[sidx], rows, sem)

    def _unpack(pidx, sidx, didx):
        for g in range(CHUNK // LANES):
            p = pidx[pl.ds(g * LANES, LANES)]
            sidx[pl.ds(g * LANES, LANES)] = p >> 14
            didx[pl.ds(g * LANES, LANES)] = p & 16383

    _pload(0, pidx0, psem0).start()
    _pload(1, pidx1, psem1).start()
    _pload(0, pidx0, psem0).wait()
    _unpack(pidx0, sidx0, didx0)
    _gather(sidx0, rows0, sem0).start()
    _pload(1, pidx1, psem1).wait()
    _unpack(pidx1, sidx1, didx1)
    _gather(sidx1, rows1, sem1).start()
    _pload(2, pidx0, psem0).start()
    _pload(3, pidx1, psem1).start()

    @pl.loop(0, CPW // 2)
    def _(j):
        i0 = 2 * j
        for i, pidx, sidx, didx, rows, sem, psem in (
                (i0, pidx0, sidx0, didx0, rows0, sem0, psem0),
                (i0 + 1, pidx1, sidx1, didx1, rows1, sem1, psem1)):
            _gather(sidx, rows, sem).wait()
            pltpu.sync_copy(rows, agg_sh.at[didx], add=True)
            if with_deg:
                pltpu.sync_copy(ones_v, deg_sh.at[didx], add=True)
            _pload(i, pidx, psem).wait()
            _unpack(pidx, sidx, didx)
            _gather(sidx, rows, sem).start()
            _pload(jnp.minimum(i + 4, CPW - 1), pidx, psem).start()

    _gather(sidx0, rows0, sem0).wait()
    _gather(sidx1, rows1, sem1).wait()
    _pload(0, pidx0, psem0).wait()
    _pload(0, pidx1, psem1).wait()

    plsc.subcore_barrier()

    pltpu.sync_copy(agg_sh.at[pl.ds(r0, RPS)],
                    agg_out.at[cid, pl.ds(r0, RPS)])
    if with_deg:
        pltpu.sync_copy(deg_sh.at[pl.ds(r0, RPS)],
                        deg_out.at[cid, pl.ds(r0, RPS)])


_AGG_OUT = jax.ShapeDtypeStruct((NC, NP, D), jnp.float32)
_DEG_OUT = jax.ShapeDtypeStruct((NC, NP, LANES), jnp.float32)

_sc_agg_deg = pl.kernel(
    functools.partial(_sc_agg_body, True),
    out_type=[_AGG_OUT, _DEG_OUT],
    mesh=_mesh,
    scratch_types=[
        pltpu.VMEM((CHUNK,), jnp.int32),
        pltpu.VMEM((CHUNK,), jnp.int32),
        pltpu.VMEM((CHUNK,), jnp.int32),
        pltpu.VMEM((CHUNK,), jnp.int32),
        pltpu.VMEM((CHUNK,), jnp.int32),
        pltpu.VMEM((CHUNK,), jnp.int32),
        pltpu.VMEM((CHUNK, D), jnp.float32),
        pltpu.VMEM((CHUNK, D), jnp.float32),
        pltpu.VMEM((CHUNK, LANES), jnp.float32),
        pltpu.VMEM((CHUNK, LANES), jnp.float32),
        pltpu.VMEM_SHARED((NP, D), jnp.float32),
        pltpu.VMEM_SHARED((NP, LANES), jnp.float32),
        pltpu.SemaphoreType.DMA,
        pltpu.SemaphoreType.DMA,
        pltpu.SemaphoreType.DMA,
        pltpu.SemaphoreType.DMA,
    ],
    compiler_params=_sc_params,
    name="sc_agg_deg",
)

_sc_agg = pl.kernel(
    functools.partial(_sc_agg_body, False),
    out_type=_AGG_OUT,
    mesh=_mesh,
    scratch_types=[
        pltpu.VMEM((CHUNK,), jnp.int32),
        pltpu.VMEM((CHUNK,), jnp.int32),
        pltpu.VMEM((CHUNK,), jnp.int32),
        pltpu.VMEM((CHUNK,), jnp.int32),
        pltpu.VMEM((CHUNK,), jnp.int32),
        pltpu.VMEM((CHUNK,), jnp.int32),
        pltpu.VMEM((CHUNK, D), jnp.float32),
        pltpu.VMEM((CHUNK, D), jnp.float32),
        pltpu.VMEM_SHARED((NP, D), jnp.float32),
        pltpu.SemaphoreType.DMA,
        pltpu.SemaphoreType.DMA,
        pltpu.SemaphoreType.DMA,
        pltpu.SemaphoreType.DMA,
    ],
    compiler_params=_sc_params,
    name="sc_agg",
)


_BLK = 2000


def _combine_body(relu, x_ref, agg_ref, deg_ref, ws_ref, wn_ref, b_ref,
                  o_ref):
    agg = agg_ref[0] + agg_ref[1]
    deg = deg_ref[0, :, 0:1] + deg_ref[1, :, 0:1]
    hn = agg / jnp.maximum(deg, 1.0)
    h = (jnp.dot(x_ref[...], ws_ref[...], preferred_element_type=jnp.float32)
         + jnp.dot(hn, wn_ref[...], preferred_element_type=jnp.float32)
         + b_ref[...])
    o_ref[...] = jnp.maximum(h, 0.0) if relu else h


def _combine(x, agg, deg, w_self, w_neigh, b, relu):
    return pl.pallas_call(
        functools.partial(_combine_body, relu),
        grid=(N // _BLK,),
        in_specs=[
            pl.BlockSpec((_BLK, D), lambda i: (i, 0)),
            pl.BlockSpec((NC, _BLK, D), lambda i: (0, i, 0)),
            pl.BlockSpec((NC, _BLK, LANES), lambda i: (0, i, 0)),
            pl.BlockSpec((D, D), lambda i: (0, 0)),
            pl.BlockSpec((D, D), lambda i: (0, 0)),
            pl.BlockSpec((1, D), lambda i: (0, 0)),
        ],
        out_specs=pl.BlockSpec((_BLK, D), lambda i: (i, 0)),
        out_shape=jax.ShapeDtypeStruct((N, D), jnp.float32),
    )(x, agg, deg, w_self, w_neigh, b.reshape(1, D))


def kernel(in_feat, edge_index, W_self1, W_neigh1, b1, W_self2, W_neigh2,
           b2):
    src = edge_index[0].astype(jnp.int32)
    dst = edge_index[1].astype(jnp.int32)
    pad = EP - E
    src_p = jnp.concatenate([src, jnp.zeros((pad,), jnp.int32)])
    dst_p = jnp.concatenate([dst, jnp.full((pad,), N, jnp.int32)])
    pidx = ((src_p << 14) | dst_p).reshape(NW, CPW, CHUNK)

    agg1, deg = _sc_agg_deg(in_feat, pidx)
    h1 = _combine(in_feat, agg1, deg, W_self1, W_neigh1, b1, relu=True)
    agg2 = _sc_agg(h1, pidx)
    return _combine(h1, agg2, deg, W_self2, W_neigh2, b2, relu=False)


# speedup vs baseline: 1.6598x; 1.6598x over previous
import functools

import jax
import jax.numpy as jnp
import numpy as np
from jax import lax
from jax.experimental import pallas as pl
from jax.experimental.pallas import tpu as pltpu
from jax.experimental.pallas import tpu_sc as plsc

N = 10000
E = 320000
D = 128

NC = 2
NS = 16
NW = NC * NS
LANES = 16

CHUNK = 128
CPW = 80
PW = CPW * CHUNK
EP = PW * NW

RPS = 632
NP = RPS * NS

_mesh = plsc.VectorSubcoreMesh(core_axis_name="c", subcore_axis_name="s")
_sc_params = pltpu.CompilerParams(use_tc_tiling_on_sc=False,
                                  needs_layout_passes=False)


def _sc_agg_body(with_deg, x_hbm, pidx_hbm, *refs):
    if with_deg:
        (agg_out, deg_out, pidx0, pidx1, sidx0, sidx1, didx0, didx1,
         rows0, rows1, rowsf, ones_v, zdeg, agg_sh, deg_sh,
         sem0, sem1, psem0, psem1) = refs
    else:
        (agg_out, pidx0, pidx1, sidx0, sidx1, didx0, didx1,
         rows0, rows1, rowsf, agg_sh, sem0, sem1, psem0, psem1) = refs

    cid = lax.axis_index("c")
    sid = lax.axis_index("s")
    wid = cid * NS + sid

    zero16 = jnp.zeros((LANES,), jnp.float32)

    @pl.loop(0, CHUNK)
    def _(r):
        @pl.loop(0, D // LANES)
        def _(g):
            rowsf[r, pl.ds(g * LANES, LANES)] = zero16

    if with_deg:
        @pl.loop(0, zdeg.shape[0])
        def _(r):
            zdeg[r, pl.ds(0, LANES)] = zero16

        ones16 = jnp.ones((LANES,), jnp.float32)

        @pl.loop(0, CHUNK)
        def _(r):
            ones_v[r, pl.ds(0, LANES)] = ones16

    r0 = sid * RPS
    nfull, rem = RPS // CHUNK, RPS % CHUNK

    for j in range(nfull):
        pltpu.sync_copy(rowsf, agg_sh.at[pl.ds(r0 + j * CHUNK, CHUNK)])
    if rem:
        pltpu.sync_copy(rowsf.at[pl.ds(0, rem)],
                        agg_sh.at[pl.ds(r0 + nfull * CHUNK, rem)])
    if with_deg:
        zn, zrem = RPS // zdeg.shape[0], RPS % zdeg.shape[0]
        for j in range(zn):
            pltpu.sync_copy(
                zdeg, deg_sh.at[pl.ds(r0 + j * zdeg.shape[0],
                                      zdeg.shape[0])])
        if zrem:
            pltpu.sync_copy(zdeg.at[pl.ds(0, zrem)],
                            deg_sh.at[pl.ds(r0 + zn * zdeg.shape[0], zrem)])

    plsc.subcore_barrier()

    def _pload(i, pidx, psem):
        return pltpu.make_async_copy(pidx_hbm.at[wid, i], pidx, psem)

    def _gather(sidx, rows, sem):
        return pltpu.make_async_copy(x_hbm.at[sidx], rows, sem)

    def _unpack(pidx, sidx, didx):
        for g in range(CHUNK // LANES):
            p = pidx[pl.ds(g * LANES, LANES)]
            sidx[pl.ds(g * LANES, LANES)] = p >> 14
            didx[pl.ds(g * LANES, LANES)] = p & 16383

    _pload(0, pidx0, psem0).start()
    _pload(1, pidx1, psem1).start()
    _pload(0, pidx0, psem0).wait()
    _unpack(pidx0, sidx0, didx0)
    _gather(sidx0, rows0, sem0).start()
    _pload(1, pidx1, psem1).wait()
    _unpack(pidx1, sidx1, didx1)
    _gather(sidx1, rows1, sem1).start()
    _pload(2, pidx0, psem0).start()
    _pload(3, pidx1, psem1).start()

    @pl.loop(0, CPW // 2)
    def _(j):
        i0 = 2 * j
        for i, pidx, sidx, didx, rows, sem, psem in (
                (i0, pidx0, sidx0, didx0, rows0, sem0, psem0),
                (i0 + 1, pidx1, sidx1, didx1, rows1, sem1, psem1)):
            _gather(sidx, rows, sem).wait()

            @pl.loop(0, CHUNK)
            def _(r):
                for g in range(D // 32):
                    w = plsc.bitcast(rows[r, pl.ds(32 * g, 32)], jnp.int32)
                    rowsf[r, pl.ds(32 * g, LANES)] = plsc.bitcast(
                        w << 16, jnp.float32)
                    rowsf[r, pl.ds(32 * g + LANES, LANES)] = plsc.bitcast(
                        w & (-65536), jnp.float32)

            pltpu.sync_copy(rowsf, agg_sh.at[didx], add=True)
            if with_deg:
                pltpu.sync_copy(ones_v, deg_sh.at[didx], add=True)
            _pload(i, pidx, psem).wait()
            _unpack(pidx, sidx, didx)
            _gather(sidx, rows, sem).start()
            _pload(jnp.minimum(i + 4, CPW - 1), pidx, psem).start()

    _gather(sidx0, rows0, sem0).wait()
    _gather(sidx1, rows1, sem1).wait()
    _pload(0, pidx0, psem0).wait()
    _pload(0, pidx1, psem1).wait()

    plsc.subcore_barrier()

    pltpu.sync_copy(agg_sh.at[pl.ds(r0, RPS)],
                    agg_out.at[cid, pl.ds(r0, RPS)])
    if with_deg:
        pltpu.sync_copy(deg_sh.at[pl.ds(r0, RPS)],
                        deg_out.at[cid, pl.ds(r0, RPS)])


_AGG_OUT = jax.ShapeDtypeStruct((NC, NP, D), jnp.float32)
_DEG_OUT = jax.ShapeDtypeStruct((NC, NP, LANES), jnp.float32)

_sc_agg_deg = pl.kernel(
    functools.partial(_sc_agg_body, True),
    out_type=[_AGG_OUT, _DEG_OUT],
    mesh=_mesh,
    scratch_types=[
        pltpu.VMEM((CHUNK,), jnp.int32),
        pltpu.VMEM((CHUNK,), jnp.int32),
        pltpu.VMEM((CHUNK,), jnp.int32),
        pltpu.VMEM((CHUNK,), jnp.int32),
        pltpu.VMEM((CHUNK,), jnp.int32),
        pltpu.VMEM((CHUNK,), jnp.int32),
        pltpu.VMEM((CHUNK, D), jnp.bfloat16),
        pltpu.VMEM((CHUNK, D), jnp.bfloat16),
        pltpu.VMEM((CHUNK, D), jnp.float32),
        pltpu.VMEM((CHUNK, LANES), jnp.float32),
        pltpu.VMEM((CHUNK, LANES), jnp.float32),
        pltpu.VMEM_SHARED((NP, D), jnp.float32),
        pltpu.VMEM_SHARED((NP, LANES), jnp.float32),
        pltpu.SemaphoreType.DMA,
        pltpu.SemaphoreType.DMA,
        pltpu.SemaphoreType.DMA,
        pltpu.SemaphoreType.DMA,
    ],
    compiler_params=_sc_params,
    name="sc_agg_deg",
)

_sc_agg = pl.kernel(
    functools.partial(_sc_agg_body, False),
    out_type=_AGG_OUT,
    mesh=_mesh,
    scratch_types=[
        pltpu.VMEM((CHUNK,), jnp.int32),
        pltpu.VMEM((CHUNK,), jnp.int32),
        pltpu.VMEM((CHUNK,), jnp.int32),
        pltpu.VMEM((CHUNK,), jnp.int32),
        pltpu.VMEM((CHUNK,), jnp.int32),
        pltpu.VMEM((CHUNK,), jnp.int32),
        pltpu.VMEM((CHUNK, D), jnp.bfloat16),
        pltpu.VMEM((CHUNK, D), jnp.bfloat16),
        pltpu.VMEM((CHUNK, D), jnp.float32),
        pltpu.VMEM_SHARED((NP, D), jnp.float32),
        pltpu.SemaphoreType.DMA,
        pltpu.SemaphoreType.DMA,
        pltpu.SemaphoreType.DMA,
        pltpu.SemaphoreType.DMA,
    ],
    compiler_params=_sc_params,
    name="sc_agg",
)


_BLK = 2000


def _combine_body(relu, emit_bf16, x_ref, agg_ref, deg_ref, ws_ref, wn_ref,
                  b_ref, *o_refs):
    agg = agg_ref[0] + agg_ref[1]
    deg = deg_ref[0, :, 0:1] + deg_ref[1, :, 0:1]
    hn = agg / jnp.maximum(deg, 1.0)
    h = (jnp.dot(x_ref[...], ws_ref[...], preferred_element_type=jnp.float32)
         + jnp.dot(hn, wn_ref[...], preferred_element_type=jnp.float32)
         + b_ref[...])
    if relu:
        h = jnp.maximum(h, 0.0)
    o_refs[0][...] = h
    if emit_bf16:
        o_refs[1][...] = h.astype(jnp.bfloat16)


def _combine(x, agg, deg, w_self, w_neigh, b, relu, emit_bf16=False):
    out_shape = [jax.ShapeDtypeStruct((N, D), jnp.float32)]
    out_specs = [pl.BlockSpec((_BLK, D), lambda i: (i, 0))]
    if emit_bf16:
        out_shape.append(jax.ShapeDtypeStruct((N, D), jnp.bfloat16))
        out_specs.append(pl.BlockSpec((_BLK, D), lambda i: (i, 0)))
    out = pl.pallas_call(
        functools.partial(_combine_body, relu, emit_bf16),
        grid=(N // _BLK,),
        in_specs=[
            pl.BlockSpec((_BLK, D), lambda i: (i, 0)),
            pl.BlockSpec((NC, _BLK, D), lambda i: (0, i, 0)),
            pl.BlockSpec((NC, _BLK, LANES), lambda i: (0, i, 0)),
            pl.BlockSpec((D, D), lambda i: (0, 0)),
            pl.BlockSpec((D, D), lambda i: (0, 0)),
            pl.BlockSpec((1, D), lambda i: (0, 0)),
        ],
        out_specs=out_specs,
        out_shape=out_shape,
    )(x, agg, deg, w_self, w_neigh, b.reshape(1, D))
    return out if emit_bf16 else out[0]


_Q = np.empty((D,), np.int32)
for _g in range(D // 32):
    for _k in range(16):
        _Q[32 * _g + _k] = 32 * _g + 2 * _k
        _Q[32 * _g + 16 + _k] = 32 * _g + 2 * _k + 1


def kernel(in_feat, edge_index, W_self1, W_neigh1, b1, W_self2, W_neigh2,
           b2):
    src = edge_index[0].astype(jnp.int32)
    dst = edge_index[1].astype(jnp.int32)
    pad = EP - E
    src_p = jnp.concatenate([src, jnp.zeros((pad,), jnp.int32)])
    dst_p = jnp.concatenate([dst, jnp.full((pad,), N, jnp.int32)])
    pidx = ((src_p << 14) | dst_p).reshape(NW, CPW, CHUNK)

    xb = in_feat.astype(jnp.bfloat16)
    wn1p = W_neigh1[_Q]
    wn2p = W_neigh2[_Q]

    agg1, deg = _sc_agg_deg(xb, pidx)
    h1, h1b = _combine(in_feat, agg1, deg, W_self1, wn1p, b1, relu=True,
                       emit_bf16=True)
    agg2 = _sc_agg(h1b, pidx)
    return _combine(h1, agg2, deg, W_self2, wn2p, b2, relu=False)


# speedup vs baseline: 1.7185x; 1.0354x over previous
import functools

import jax
import jax.numpy as jnp
import numpy as np
from jax import lax
from jax.experimental import pallas as pl
from jax.experimental.pallas import tpu as pltpu
from jax.experimental.pallas import tpu_sc as plsc

N = 10000
E = 320000
D = 128

NC = 2
NS = 16
NW = NC * NS
LANES = 16

CHUNK = 128
CPW = 80
PW = CPW * CHUNK
EP = PW * NW

RPS = 632
NP = RPS * NS

_mesh = plsc.VectorSubcoreMesh(core_axis_name="c", subcore_axis_name="s")
_sc_params = pltpu.CompilerParams(use_tc_tiling_on_sc=False,
                                  needs_layout_passes=False)


QR = 32


def _sc_agg_body(with_deg, x_hbm, pidx_hbm, *refs):
    if with_deg:
        (agg_out, deg_out, pidx_all, sidx0, sidx1, didx0, didx1,
         rows0, rows1, rowsf0, rowsf1, ones_v, zdeg, agg_sh, deg_sh,
         sem0, sem1, ssem0, ssem1) = refs
    else:
        (agg_out, pidx_all, sidx0, sidx1, didx0, didx1,
         rows0, rows1, rowsf0, rowsf1, agg_sh,
         sem0, sem1, ssem0, ssem1) = refs

    cid = lax.axis_index("c")
    sid = lax.axis_index("s")
    wid = cid * NS + sid

    zero16 = jnp.zeros((LANES,), jnp.float32)

    @pl.loop(0, QR)
    def _(r):
        @pl.loop(0, D // LANES)
        def _(g):
            rowsf0[r, pl.ds(g * LANES, LANES)] = zero16

    if with_deg:
        @pl.loop(0, zdeg.shape[0])
        def _(r):
            zdeg[r, pl.ds(0, LANES)] = zero16

        ones16 = jnp.ones((LANES,), jnp.float32)

        @pl.loop(0, QR)
        def _(r):
            ones_v[r, pl.ds(0, LANES)] = ones16

    r0 = sid * RPS
    nfull, rem = RPS // QR, RPS % QR

    for j in range(nfull):
        pltpu.sync_copy(rowsf0, agg_sh.at[pl.ds(r0 + j * QR, QR)])
    if rem:
        pltpu.sync_copy(rowsf0.at[pl.ds(0, rem)],
                        agg_sh.at[pl.ds(r0 + nfull * QR, rem)])
    if with_deg:
        zn, zrem = RPS // zdeg.shape[0], RPS % zdeg.shape[0]
        for j in range(zn):
            pltpu.sync_copy(
                zdeg, deg_sh.at[pl.ds(r0 + j * zdeg.shape[0],
                                      zdeg.shape[0])])
        if zrem:
            pltpu.sync_copy(zdeg.at[pl.ds(0, zrem)],
                            deg_sh.at[pl.ds(r0 + zn * zdeg.shape[0], zrem)])

    pltpu.sync_copy(pidx_hbm.at[wid], pidx_all)

    plsc.subcore_barrier()

    def _gather(sidx, rows, sem):
        return pltpu.make_async_copy(x_hbm.at[sidx], rows, sem)

    def _unpack(i, sidx, didx):
        for g in range(CHUNK // LANES):
            p = pidx_all[i, pl.ds(g * LANES, LANES)]
            sidx[pl.ds(g * LANES, LANES)] = p >> 14
            didx[g // 2, pl.ds((g % 2) * LANES, LANES)] = p & 16383

    def _convert(rows, q, rowsf):
        @pl.loop(0, QR)
        def _(r):
            for g in range(D // 32):
                w = plsc.bitcast(rows[q * QR + r, pl.ds(32 * g, 32)],
                                 jnp.int32)
                rowsf[r, pl.ds(32 * g, LANES)] = plsc.bitcast(
                    w << 16, jnp.float32)
                rowsf[r, pl.ds(32 * g + LANES, LANES)] = plsc.bitcast(
                    w & (-65536), jnp.float32)

    def _scat(rowsf, didx, q, ssem):
        return pltpu.make_async_copy(rowsf, agg_sh.at[didx.at[q]], ssem)

    _unpack(0, sidx0, didx0)
    _gather(sidx0, rows0, sem0).start()
    _unpack(1, sidx1, didx1)
    _gather(sidx1, rows1, sem1).start()

    @pl.loop(0, CPW // 2)
    def _(j):
        i0 = 2 * j
        for i, sidx, didx, rows, sem in (
                (i0, sidx0, didx0, rows0, sem0),
                (i0 + 1, sidx1, didx1, rows1, sem1)):
            _gather(sidx, rows, sem).wait()

            _convert(rows, 0, rowsf0)
            _scat(rowsf0, didx, 0, ssem0).start()
            _convert(rows, 1, rowsf1)
            _scat(rowsf1, didx, 1, ssem1).start()
            _scat(rowsf0, didx, 0, ssem0).wait()
            _convert(rows, 2, rowsf0)
            _scat(rowsf0, didx, 2, ssem0).start()
            _scat(rowsf1, didx, 1, ssem1).wait()
            _convert(rows, 3, rowsf1)
            _scat(rowsf1, didx, 3, ssem1).start()
            if with_deg:
                for q in range(4):
                    pltpu.sync_copy(ones_v, deg_sh.at[didx.at[q]], add=True)
            _scat(rowsf0, didx, 0, ssem0).wait()
            _scat(rowsf1, didx, 1, ssem1).wait()

            _unpack(jnp.minimum(i + 2, CPW - 1), sidx, didx)
            _gather(sidx, rows, sem).start()

    _gather(sidx0, rows0, sem0).wait()
    _gather(sidx1, rows1, sem1).wait()

    plsc.subcore_barrier()

    pltpu.sync_copy(agg_sh.at[pl.ds(r0, RPS)],
                    agg_out.at[cid, pl.ds(r0, RPS)])
    if with_deg:
        pltpu.sync_copy(deg_sh.at[pl.ds(r0, RPS)],
                        deg_out.at[cid, pl.ds(r0, RPS)])


_AGG_OUT = jax.ShapeDtypeStruct((NC, NP, D), jnp.float32)
_DEG_OUT = jax.ShapeDtypeStruct((NC, NP, LANES), jnp.float32)

_sc_agg_deg = pl.kernel(
    functools.partial(_sc_agg_body, True),
    out_type=[_AGG_OUT, _DEG_OUT],
    mesh=_mesh,
    scratch_types=[
        pltpu.VMEM((CPW, CHUNK), jnp.int32),
        pltpu.VMEM((CHUNK,), jnp.int32),
        pltpu.VMEM((CHUNK,), jnp.int32),
        pltpu.VMEM((4, QR), jnp.int32),
        pltpu.VMEM((4, QR), jnp.int32),
        pltpu.VMEM((CHUNK, D), jnp.bfloat16),
        pltpu.VMEM((CHUNK, D), jnp.bfloat16),
        pltpu.VMEM((QR, D), jnp.float32),
        pltpu.VMEM((QR, D), jnp.float32),
        pltpu.VMEM((QR, LANES), jnp.float32),
        pltpu.VMEM((64, LANES), jnp.float32),
        pltpu.VMEM_SHARED((NP, D), jnp.float32),
        pltpu.VMEM_SHARED((NP, LANES), jnp.float32),
        pltpu.SemaphoreType.DMA,
        pltpu.SemaphoreType.DMA,
        pltpu.SemaphoreType.DMA,
        pltpu.SemaphoreType.DMA,
    ],
    compiler_params=_sc_params,
    name="sc_agg_deg",
)

_sc_agg = pl.kernel(
    functools.partial(_sc_agg_body, False),
    out_type=_AGG_OUT,
    mesh=_mesh,
    scratch_types=[
        pltpu.VMEM((CPW, CHUNK), jnp.int32),
        pltpu.VMEM((CHUNK,), jnp.int32),
        pltpu.VMEM((CHUNK,), jnp.int32),
        pltpu.VMEM((4, QR), jnp.int32),
        pltpu.VMEM((4, QR), jnp.int32),
        pltpu.VMEM((CHUNK, D), jnp.bfloat16),
        pltpu.VMEM((CHUNK, D), jnp.bfloat16),
        pltpu.VMEM((QR, D), jnp.float32),
        pltpu.VMEM((QR, D), jnp.float32),
        pltpu.VMEM_SHARED((NP, D), jnp.float32),
        pltpu.SemaphoreType.DMA,
        pltpu.SemaphoreType.DMA,
        pltpu.SemaphoreType.DMA,
        pltpu.SemaphoreType.DMA,
    ],
    compiler_params=_sc_params,
    name="sc_agg",
)


_BLK = 2000


def _combine_body(relu, emit_bf16, x_ref, agg_ref, deg_ref, ws_ref, wn_ref,
                  b_ref, *o_refs):
    agg = agg_ref[0] + agg_ref[1]
    deg = deg_ref[0, :, 0:1] + deg_ref[1, :, 0:1]
    hn = agg / jnp.maximum(deg, 1.0)
    h = (jnp.dot(x_ref[...], ws_ref[...], preferred_element_type=jnp.float32)
         + jnp.dot(hn, wn_ref[...], preferred_element_type=jnp.float32)
         + b_ref[...])
    if relu:
        h = jnp.maximum(h, 0.0)
    o_refs[0][...] = h
    if emit_bf16:
        o_refs[1][...] = h.astype(jnp.bfloat16)


def _combine(x, agg, deg, w_self, w_neigh, b, relu, emit_bf16=False):
    out_shape = [jax.ShapeDtypeStruct((N, D), jnp.float32)]
    out_specs = [pl.BlockSpec((_BLK, D), lambda i: (i, 0))]
    if emit_bf16:
        out_shape.append(jax.ShapeDtypeStruct((N, D), jnp.bfloat16))
        out_specs.append(pl.BlockSpec((_BLK, D), lambda i: (i, 0)))
    out = pl.pallas_call(
        functools.partial(_combine_body, relu, emit_bf16),
        grid=(N // _BLK,),
        in_specs=[
            pl.BlockSpec((_BLK, D), lambda i: (i, 0)),
            pl.BlockSpec((NC, _BLK, D), lambda i: (0, i, 0)),
            pl.BlockSpec((NC, _BLK, LANES), lambda i: (0, i, 0)),
            pl.BlockSpec((D, D), lambda i: (0, 0)),
            pl.BlockSpec((D, D), lambda i: (0, 0)),
            pl.BlockSpec((1, D), lambda i: (0, 0)),
        ],
        out_specs=out_specs,
        out_shape=out_shape,
    )(x, agg, deg, w_self, w_neigh, b.reshape(1, D))
    return out if emit_bf16 else out[0]


_Q = np.empty((D,), np.int32)
for _g in range(D // 32):
    for _k in range(16):
        _Q[32 * _g + _k] = 32 * _g + 2 * _k
        _Q[32 * _g + 16 + _k] = 32 * _g + 2 * _k + 1


def kernel(in_feat, edge_index, W_self1, W_neigh1, b1, W_self2, W_neigh2,
           b2):
    src = edge_index[0].astype(jnp.int32)
    dst = edge_index[1].astype(jnp.int32)
    pad = EP - E
    src_p = jnp.concatenate([src, jnp.zeros((pad,), jnp.int32)])
    dst_p = jnp.concatenate([dst, jnp.full((pad,), N, jnp.int32)])
    pidx = ((src_p << 14) | dst_p).reshape(NW, CPW, CHUNK)

    xb = in_feat.astype(jnp.bfloat16)
    wn1p = W_neigh1[_Q]
    wn2p = W_neigh2[_Q]

    agg1, deg = _sc_agg_deg(xb, pidx)
    h1, h1b = _combine(in_feat, agg1, deg, W_self1, wn1p, b1, relu=True,
                       emit_bf16=True)
    agg2 = _sc_agg(h1b, pidx)
    return _combine(h1, agg2, deg, W_self2, wn2p, b2, relu=False)


# speedup vs baseline: 1.7841x; 1.0382x over previous
import functools

import jax
import jax.numpy as jnp
import numpy as np
from jax import lax
from jax.experimental import pallas as pl
from jax.experimental.pallas import tpu as pltpu
from jax.experimental.pallas import tpu_sc as plsc

N = 10000
E = 320000
D = 128

NC = 2
NS = 16
NW = NC * NS
LANES = 16

CHUNK = 128
CPW = 80
PW = CPW * CHUNK
EP = PW * NW

RPS = 632
NP = RPS * NS

_mesh = plsc.VectorSubcoreMesh(core_axis_name="c", subcore_axis_name="s")
_sc_params = pltpu.CompilerParams(use_tc_tiling_on_sc=False,
                                  needs_layout_passes=False)


QR = 32


def _sc_agg_body(with_deg, x_hbm, pidx_hbm, *refs):
    if with_deg:
        (agg_out, deg_out, sidx0, sidx1, didx0, didx1,
         rows0, rows1, rowsf0, rowsf1, ones_v, zdeg, pidx0, pidx1,
         agg_sh, deg_sh, sem0, sem1, ssem0, ssem1, psem0, psem1) = refs
    else:
        (agg_out, sidx0, sidx1, didx0, didx1,
         rows0, rows1, rowsf0, rowsf1, pidx0, pidx1, agg_sh,
         sem0, sem1, ssem0, ssem1, psem0, psem1) = refs

    cid = lax.axis_index("c")
    sid = lax.axis_index("s")
    wid = cid * NS + sid

    zero16 = jnp.zeros((LANES,), jnp.float32)

    @pl.loop(0, QR)
    def _(r):
        @pl.loop(0, D // LANES)
        def _(g):
            rowsf0[r, pl.ds(g * LANES, LANES)] = zero16

    if with_deg:
        @pl.loop(0, zdeg.shape[0])
        def _(r):
            zdeg[r, pl.ds(0, LANES)] = zero16

        ones16 = jnp.ones((LANES,), jnp.float32)

        @pl.loop(0, QR)
        def _(r):
            ones_v[r, pl.ds(0, LANES)] = ones16

    r0 = sid * RPS
    nfull, rem = RPS // QR, RPS % QR

    for j in range(nfull):
        pltpu.sync_copy(rowsf0, agg_sh.at[pl.ds(r0 + j * QR, QR)])
    if rem:
        pltpu.sync_copy(rowsf0.at[pl.ds(0, rem)],
                        agg_sh.at[pl.ds(r0 + nfull * QR, rem)])
    if with_deg:
        zn, zrem = RPS // zdeg.shape[0], RPS % zdeg.shape[0]
        for j in range(zn):
            pltpu.sync_copy(
                zdeg, deg_sh.at[pl.ds(r0 + j * zdeg.shape[0],
                                      zdeg.shape[0])])
        if zrem:
            pltpu.sync_copy(zdeg.at[pl.ds(0, zrem)],
                            deg_sh.at[pl.ds(r0 + zn * zdeg.shape[0], zrem)])

    plsc.subcore_barrier()

    def _pload(i, pidx, psem):
        return pltpu.make_async_copy(pidx_hbm.at[wid, i], pidx, psem)

    def _gather(sidx, rows, sem):
        return pltpu.make_async_copy(x_hbm.at[sidx], rows, sem)

    def _unpack(pidx, sidx, didx):
        for g in range(CHUNK // LANES):
            p = pidx[pl.ds(g * LANES, LANES)]
            sidx[pl.ds(g * LANES, LANES)] = p >> 14
            didx[g // 2, pl.ds((g % 2) * LANES, LANES)] = p & 16383

    def _convert(rows, q, rowsf):
        @pl.loop(0, QR)
        def _(r):
            for g in range(D // 32):
                w = plsc.bitcast(rows[q * QR + r, pl.ds(32 * g, 32)],
                                 jnp.int32)
                rowsf[r, pl.ds(32 * g, LANES)] = plsc.bitcast(
                    w << 16, jnp.float32)
                rowsf[r, pl.ds(32 * g + LANES, LANES)] = plsc.bitcast(
                    w & (-65536), jnp.float32)

    def _scat_start(rowsf, didx, q, ssem):
        pltpu.async_copy(rowsf, agg_sh.at[didx.at[q]], ssem, add=True)

    def _scat_wait(rowsf, didx, q, ssem):
        pltpu.make_async_copy(rowsf, agg_sh.at[didx.at[q]], ssem).wait()

    _pload(0, pidx0, psem0).start()
    _pload(1, pidx1, psem1).start()
    _pload(0, pidx0, psem0).wait()
    _unpack(pidx0, sidx0, didx0)
    _gather(sidx0, rows0, sem0).start()
    _pload(1, pidx1, psem1).wait()
    _unpack(pidx1, sidx1, didx1)
    _gather(sidx1, rows1, sem1).start()
    _pload(2, pidx0, psem0).start()
    _pload(3, pidx1, psem1).start()

    @pl.loop(0, CPW // 2)
    def _(j):
        i0 = 2 * j
        for i, pidx, sidx, didx, rows, sem, psem in (
                (i0, pidx0, sidx0, didx0, rows0, sem0, psem0),
                (i0 + 1, pidx1, sidx1, didx1, rows1, sem1, psem1)):
            _gather(sidx, rows, sem).wait()

            _convert(rows, 0, rowsf0)
            _scat_start(rowsf0, didx, 0, ssem0)
            _convert(rows, 1, rowsf1)
            _scat_start(rowsf1, didx, 1, ssem1)
            _scat_wait(rowsf0, didx, 0, ssem0)
            _convert(rows, 2, rowsf0)
            _scat_start(rowsf0, didx, 2, ssem0)
            _scat_wait(rowsf1, didx, 1, ssem1)
            _convert(rows, 3, rowsf1)
            _scat_start(rowsf1, didx, 3, ssem1)
            if with_deg:
                for q in range(4):
                    pltpu.sync_copy(ones_v, deg_sh.at[didx.at[q]], add=True)
            _scat_wait(rowsf0, didx, 2, ssem0)
            _scat_wait(rowsf1, didx, 3, ssem1)

            _pload(i, pidx, psem).wait()
            _unpack(pidx, sidx, didx)
            _gather(sidx, rows, sem).start()
            _pload(jnp.minimum(i + 4, CPW - 1), pidx, psem).start()

    _gather(sidx0, rows0, sem0).wait()
    _gather(sidx1, rows1, sem1).wait()
    _pload(0, pidx0, psem0).wait()
    _pload(0, pidx1, psem1).wait()

    plsc.subcore_barrier()

    pltpu.sync_copy(agg_sh.at[pl.ds(r0, RPS)],
                    agg_out.at[cid, pl.ds(r0, RPS)])
    if with_deg:
        pltpu.sync_copy(deg_sh.at[pl.ds(r0, RPS)],
                        deg_out.at[cid, pl.ds(r0, RPS)])


_AGG_OUT = jax.ShapeDtypeStruct((NC, NP, D), jnp.float32)
_DEG_OUT = jax.ShapeDtypeStruct((NC, NP, LANES), jnp.float32)

_sc_agg_deg = pl.kernel(
    functools.partial(_sc_agg_body, True),
    out_type=[_AGG_OUT, _DEG_OUT],
    mesh=_mesh,
    scratch_types=[
        pltpu.VMEM((CHUNK,), jnp.int32),
        pltpu.VMEM((CHUNK,), jnp.int32),
        pltpu.VMEM((4, QR), jnp.int32),
        pltpu.VMEM((4, QR), jnp.int32),
        pltpu.VMEM((CHUNK, D), jnp.bfloat16),
        pltpu.VMEM((CHUNK, D), jnp.bfloat16),
        pltpu.VMEM((QR, D), jnp.float32),
        pltpu.VMEM((QR, D), jnp.float32),
        pltpu.VMEM((QR, LANES), jnp.float32),
        pltpu.VMEM((64, LANES), jnp.float32),
        pltpu.VMEM((CHUNK,), jnp.int32),
        pltpu.VMEM((CHUNK,), jnp.int32),
        pltpu.VMEM_SHARED((NP, D), jnp.float32),
        pltpu.VMEM_SHARED((NP, LANES), jnp.float32),
        pltpu.SemaphoreType.DMA,
        pltpu.SemaphoreType.DMA,
        pltpu.SemaphoreType.DMA,
        pltpu.SemaphoreType.DMA,
        pltpu.SemaphoreType.DMA,
        pltpu.SemaphoreType.DMA,
    ],
    compiler_params=_sc_params,
    name="sc_agg_deg",
)

_sc_agg = pl.kernel(
    functools.partial(_sc_agg_body, False),
    out_type=_AGG_OUT,
    mesh=_mesh,
    scratch_types=[
        pltpu.VMEM((CHUNK,), jnp.int32),
        pltpu.VMEM((CHUNK,), jnp.int32),
        pltpu.VMEM((4, QR), jnp.int32),
        pltpu.VMEM((4, QR), jnp.int32),
        pltpu.VMEM((CHUNK, D), jnp.bfloat16),
        pltpu.VMEM((CHUNK, D), jnp.bfloat16),
        pltpu.VMEM((QR, D), jnp.float32),
        pltpu.VMEM((QR, D), jnp.float32),
        pltpu.VMEM((CHUNK,), jnp.int32),
        pltpu.VMEM((CHUNK,), jnp.int32),
        pltpu.VMEM_SHARED((NP, D), jnp.float32),
        pltpu.SemaphoreType.DMA,
        pltpu.SemaphoreType.DMA,
        pltpu.SemaphoreType.DMA,
        pltpu.SemaphoreType.DMA,
        pltpu.SemaphoreType.DMA,
        pltpu.SemaphoreType.DMA,
    ],
    compiler_params=_sc_params,
    name="sc_agg",
)


_BLK = 2000


def _combine_body(relu, emit_bf16, x_ref, agg_ref, deg_ref, ws_ref, wn_ref,
                  b_ref, *o_refs):
    agg = agg_ref[0] + agg_ref[1]
    deg = deg_ref[0, :, 0:1] + deg_ref[1, :, 0:1]
    hn = agg / jnp.maximum(deg, 1.0)
    h = (jnp.dot(x_ref[...], ws_ref[...], preferred_element_type=jnp.float32)
         + jnp.dot(hn, wn_ref[...], preferred_element_type=jnp.float32)
         + b_ref[...])
    if relu:
        h = jnp.maximum(h, 0.0)
    o_refs[0][...] = h
    if emit_bf16:
        o_refs[1][...] = h.astype(jnp.bfloat16)


def _combine(x, agg, deg, w_self, w_neigh, b, relu, emit_bf16=False):
    out_shape = [jax.ShapeDtypeStruct((N, D), jnp.float32)]
    out_specs = [pl.BlockSpec((_BLK, D), lambda i: (i, 0))]
    if emit_bf16:
        out_shape.append(jax.ShapeDtypeStruct((N, D), jnp.bfloat16))
        out_specs.append(pl.BlockSpec((_BLK, D), lambda i: (i, 0)))
    out = pl.pallas_call(
        functools.partial(_combine_body, relu, emit_bf16),
        grid=(N // _BLK,),
        in_specs=[
            pl.BlockSpec((_BLK, D), lambda i: (i, 0)),
            pl.BlockSpec((NC, _BLK, D), lambda i: (0, i, 0)),
            pl.BlockSpec((NC, _BLK, LANES), lambda i: (0, i, 0)),
            pl.BlockSpec((D, D), lambda i: (0, 0)),
            pl.BlockSpec((D, D), lambda i: (0, 0)),
            pl.BlockSpec((1, D), lambda i: (0, 0)),
        ],
        out_specs=out_specs,
        out_shape=out_shape,
    )(x, agg, deg, w_self, w_neigh, b.reshape(1, D))
    return out if emit_bf16 else out[0]


_Q = np.empty((D,), np.int32)
for _g in range(D // 32):
    for _k in range(16):
        _Q[32 * _g + _k] = 32 * _g + 2 * _k
        _Q[32 * _g + 16 + _k] = 32 * _g + 2 * _k + 1


def kernel(in_feat, edge_index, W_self1, W_neigh1, b1, W_self2, W_neigh2,
           b2):
    src = edge_index[0].astype(jnp.int32)
    dst = edge_index[1].astype(jnp.int32)
    pad = EP - E
    src_p = jnp.concatenate([src, jnp.zeros((pad,), jnp.int32)])
    dst_p = jnp.concatenate([dst, jnp.full((pad,), N, jnp.int32)])
    pidx = ((src_p << 14) | dst_p).reshape(NW, CPW, CHUNK)

    xb = in_feat.astype(jnp.bfloat16)
    wn1p = W_neigh1[_Q]
    wn2p = W_neigh2[_Q]

    agg1, deg = _sc_agg_deg(xb, pidx)
    h1, h1b = _combine(in_feat, agg1, deg, W_self1, wn1p, b1, relu=True,
                       emit_bf16=True)
    agg2 = _sc_agg(h1b, pidx)
    return _combine(h1, agg2, deg, W_self2, wn2p, b2, relu=False)
